# Initial kernel scaffold; baseline (speedup 1.0000x reference)
import functools

import jax
import jax.numpy as jnp
from jax.experimental import pallas as pl
from jax.experimental.pallas import tpu as pltpu

_VMEM_LIMIT = 64 * 1024 * 1024


def _conv_kernel(xt_ref, xm_ref, xb_ref, w_ref, b_ref,
                 y_ref, s_ref, q_ref, *, H, W):
    xc = jnp.concatenate([xt_ref[0, 0], xm_ref[0, 0], xb_ref[0, 0]],
                         axis=-1)
    patches = [
        xc[kh:kh + H, kw:kw + W, :].reshape(H * W, xc.shape[-1])
        for kh in range(3) for kw in range(3)
    ]
    p = jnp.concatenate(patches, axis=-1)
    acc = jnp.dot(p, w_ref[...],
                  preferred_element_type=jnp.float32)
    acc = acc + b_ref[...]
    y_ref[0, 0] = acc.astype(y_ref.dtype)
    s_ref[0, 0] = jnp.sum(acc, axis=0, keepdims=True)
    q_ref[0, 0] = jnp.sum(acc * acc, axis=0, keepdims=True)


def _bn_relu_kernel(y_ref, sc_ref, sh_ref, o_ref):
    z = y_ref[...].astype(jnp.float32) * sc_ref[...] + sh_ref[...]
    o_ref[...] = jnp.maximum(z, 0.0)


def kernel(x, w, b, gamma, beta, alpha):
    N, Cin, D, H, W = x.shape
    Cout = w.shape[0]
    HW = H * W
    K = 9 * 3 * Cin

    x_cl = jnp.transpose(x, (0, 2, 3, 4, 1))
    x_pad = jnp.pad(x_cl, ((0, 0), (1, 1), (1, 1), (1, 1),
                           (0, 0))).astype(jnp.bfloat16)
    w_t = jnp.transpose(w, (3, 4, 2, 1, 0)).reshape(K, Cout).astype(jnp.bfloat16)
    b2 = b.reshape(1, Cout).astype(jnp.float32)

    def x_spec(shift):
        return pl.BlockSpec((1, 1, H + 2, W + 2, Cin),
                            lambda n, d, s=shift: (n, d + s, 0, 0, 0))

    y, s_sum, s_sq = pl.pallas_call(
        functools.partial(_conv_kernel, H=H, W=W),
        out_shape=(
            jax.ShapeDtypeStruct((N, D, HW, Cout), jnp.bfloat16),
            jax.ShapeDtypeStruct((N, D, 1, Cout), jnp.float32),
            jax.ShapeDtypeStruct((N, D, 1, Cout), jnp.float32),
        ),
        grid_spec=pltpu.PrefetchScalarGridSpec(
            num_scalar_prefetch=0,
            grid=(N, D),
            in_specs=[
                x_spec(0), x_spec(1), x_spec(2),
                pl.BlockSpec((K, Cout), lambda n, d: (0, 0)),
                pl.BlockSpec((1, Cout), lambda n, d: (0, 0)),
            ],
            out_specs=(
                pl.BlockSpec((1, 1, HW, Cout), lambda n, d: (n, d, 0, 0)),
                pl.BlockSpec((1, 1, 1, Cout), lambda n, d: (n, d, 0, 0)),
                pl.BlockSpec((1, 1, 1, Cout), lambda n, d: (n, d, 0, 0)),
            ),
        ),
        compiler_params=pltpu.CompilerParams(
            dimension_semantics=("parallel", "parallel"),
            vmem_limit_bytes=_VMEM_LIMIT,
        ),
    )(x_pad, x_pad, x_pad, w_t, b2)

    count = float(N * D * HW)
    mean = jnp.sum(s_sum, axis=(0, 1, 2)) / count
    var = jnp.maximum(jnp.sum(s_sq, axis=(0, 1, 2)) / count - mean * mean, 0.0)
    inv = gamma.astype(jnp.float32) / jnp.sqrt(var + 1e-5)
    scale = inv.reshape(1, Cout)
    shift = (beta.astype(jnp.float32) - mean * inv).reshape(1, Cout)

    rows = N * D * HW
    tr = 8192
    out2 = pl.pallas_call(
        _bn_relu_kernel,
        out_shape=jax.ShapeDtypeStruct((rows, Cout), jnp.float32),
        grid_spec=pltpu.PrefetchScalarGridSpec(
            num_scalar_prefetch=0,
            grid=(rows // tr,),
            in_specs=[
                pl.BlockSpec((tr, Cout), lambda i: (i, 0)),
                pl.BlockSpec((1, Cout), lambda i: (0, 0)),
                pl.BlockSpec((1, Cout), lambda i: (0, 0)),
            ],
            out_specs=pl.BlockSpec((tr, Cout), lambda i: (i, 0)),
        ),
        compiler_params=pltpu.CompilerParams(
            dimension_semantics=("parallel",),
            vmem_limit_bytes=_VMEM_LIMIT,
        ),
    )(y.reshape(rows, Cout), scale, shift)

    out = out2.reshape(N, D, H, W, Cout)
    return jnp.transpose(out, (0, 4, 1, 2, 3))


# speedup vs baseline: 1.6247x; 1.6247x over previous
import functools

import jax
import jax.numpy as jnp
from jax.experimental import pallas as pl
from jax.experimental.pallas import tpu as pltpu

_VMEM_LIMIT = 64 * 1024 * 1024


def _conv_kernel(xt_ref, xm_ref, xb_ref, w_ref, b_ref,
                 y_ref, s_ref, q_ref, *, H, W):
    xc = jnp.concatenate([xt_ref[0, 0], xm_ref[0, 0], xb_ref[0, 0]],
                         axis=-1)
    patches = [
        xc[kh:kh + H, kw:kw + W, :].reshape(H * W, xc.shape[-1])
        for kh in range(3) for kw in range(3)
    ]
    p = jnp.concatenate(patches, axis=-1)
    acc = jnp.dot(p, w_ref[...],
                  preferred_element_type=jnp.float32)
    acc = acc + b_ref[...]
    y_ref[0, 0] = acc.astype(y_ref.dtype)
    s_ref[0, 0] = jnp.sum(acc, axis=0, keepdims=True)
    q_ref[0, 0] = jnp.sum(acc * acc, axis=0, keepdims=True)


def _bn_relu_kernel(y_ref, sc_ref, sh_ref, o_ref):
    z = y_ref[...].astype(jnp.float32) * sc_ref[...] + sh_ref[...]
    o_ref[...] = jnp.maximum(z, 0.0)


def kernel(x, w, b, gamma, beta, alpha):
    N, Cin, D, H, W = x.shape
    Cout = w.shape[0]
    HW = H * W
    K = 9 * 3 * Cin

    x_cl = jnp.transpose(x, (0, 2, 3, 4, 1))
    x_pad = jnp.pad(x_cl, ((0, 0), (1, 1), (1, 1), (1, 1),
                           (0, 0))).astype(jnp.bfloat16)
    w_t = jnp.transpose(w, (3, 4, 2, 1, 0)).reshape(K, Cout).astype(jnp.bfloat16)
    b2 = b.reshape(1, Cout).astype(jnp.float32)

    def x_spec(shift):
        return pl.BlockSpec((1, 1, H + 2, W + 2, Cin),
                            lambda n, d, s=shift: (n, d + s, 0, 0, 0))

    y, s_sum, s_sq = pl.pallas_call(
        functools.partial(_conv_kernel, H=H, W=W),
        out_shape=(
            jax.ShapeDtypeStruct((N, D, HW, Cout), jnp.bfloat16),
            jax.ShapeDtypeStruct((N, D, 1, Cout), jnp.float32),
            jax.ShapeDtypeStruct((N, D, 1, Cout), jnp.float32),
        ),
        grid_spec=pltpu.PrefetchScalarGridSpec(
            num_scalar_prefetch=0,
            grid=(N, D),
            in_specs=[
                x_spec(0), x_spec(1), x_spec(2),
                pl.BlockSpec((K, Cout), lambda n, d: (0, 0)),
                pl.BlockSpec((1, Cout), lambda n, d: (0, 0)),
            ],
            out_specs=(
                pl.BlockSpec((1, 1, HW, Cout), lambda n, d: (n, d, 0, 0)),
                pl.BlockSpec((1, 1, 1, Cout), lambda n, d: (n, d, 0, 0)),
                pl.BlockSpec((1, 1, 1, Cout), lambda n, d: (n, d, 0, 0)),
            ),
        ),
        compiler_params=pltpu.CompilerParams(
            dimension_semantics=("parallel", "parallel"),
            vmem_limit_bytes=_VMEM_LIMIT,
        ),
    )(x_pad, x_pad, x_pad, w_t, b2)

    count = float(N * D * HW)
    mean = jnp.sum(s_sum, axis=(0, 1, 2)) / count
    var = jnp.maximum(jnp.sum(s_sq, axis=(0, 1, 2)) / count - mean * mean, 0.0)
    inv = gamma.astype(jnp.float32) / jnp.sqrt(var + 1e-5)
    scale = inv.reshape(1, Cout)
    shift = (beta.astype(jnp.float32) - mean * inv).reshape(1, Cout)

    rows = N * D * HW
    tr = 8192
    while rows % tr != 0:
        tr //= 2
    out2 = pl.pallas_call(
        _bn_relu_kernel,
        out_shape=jax.ShapeDtypeStruct((rows, Cout), jnp.float32),
        grid_spec=pltpu.PrefetchScalarGridSpec(
            num_scalar_prefetch=0,
            grid=(rows // tr,),
            in_specs=[
                pl.BlockSpec((tr, Cout), lambda i: (i, 0)),
                pl.BlockSpec((1, Cout), lambda i: (0, 0)),
                pl.BlockSpec((1, Cout), lambda i: (0, 0)),
            ],
            out_specs=pl.BlockSpec((tr, Cout), lambda i: (i, 0)),
        ),
        compiler_params=pltpu.CompilerParams(
            dimension_semantics=("parallel",),
            vmem_limit_bytes=_VMEM_LIMIT,
        ),
    )(y.reshape(rows, Cout), scale, shift)

    out = out2.reshape(N, D, H, W, Cout)
    return jnp.transpose(out, (0, 4, 1, 2, 3))


# speedup vs baseline: 3.3297x; 2.0494x over previous
import functools

import jax
import jax.numpy as jnp
from jax.experimental import pallas as pl
from jax.experimental.pallas import tpu as pltpu

_VMEM_LIMIT = 64 * 1024 * 1024


def _shift_lanes(v, k, zeros_k):
    if k > 0:
        return jnp.concatenate([v[:, k:], zeros_k], axis=1)
    if k < 0:
        return jnp.concatenate([zeros_k, v[:, :k]], axis=1)
    return v


def _conv_kernel(xt_ref, xm_ref, xb_ref, w_ref, b_ref,
                 y_ref, s_ref, q_ref, *, H, W, D):
    d = pl.program_id(1)
    C3 = 3 * xt_ref.shape[1]
    HW = H * W

    x3 = jnp.concatenate([xt_ref[0], xm_ref[0], xb_ref[0]], axis=0)
    x3 = x3.reshape(C3, HW)
    row = jax.lax.broadcasted_iota(jnp.int32, (C3, 1), 0)
    top = jnp.where(d > 0, 1.0, 0.0)
    bot = jnp.where(d < D - 1, 1.0, 0.0)
    dmask = jnp.where(row < C3 // 3, top, jnp.where(row >= 2 * (C3 // 3), bot, 1.0))
    x3 = (x3 * dmask).astype(jnp.bfloat16)

    col = jax.lax.broadcasted_iota(jnp.int32, (1, HW), 1) % W
    mask_l = jnp.where(col == 0, 0.0, 1.0).astype(jnp.bfloat16)
    mask_r = jnp.where(col == W - 1, 0.0, 1.0).astype(jnp.bfloat16)
    zeros49 = jnp.zeros((C3, 49), jnp.bfloat16)

    pieces = []
    for kh in range(3):
        for kw in range(3):
            k = (kh - 1) * W + (kw - 1)
            p = _shift_lanes(x3, k, zeros49[:, :abs(k)] if k else None)
            if kw == 0:
                p = p * mask_l
            elif kw == 2:
                p = p * mask_r
            pieces.append(p)
    pt = jnp.concatenate(pieces, axis=0)

    acc = jnp.dot(w_ref[...], pt,
                  preferred_element_type=jnp.float32)
    acc = acc + b_ref[...]
    y_ref[0, 0] = acc.reshape(y_ref.shape[2:]).astype(y_ref.dtype)
    s_ref[0, 0] = jnp.sum(acc, axis=1, keepdims=True)
    q_ref[0, 0] = jnp.sum(acc * acc, axis=1, keepdims=True)


def _bn_relu_kernel(y_ref, sc_ref, sh_ref, o_ref):
    sc = sc_ref[...].reshape(-1, 1, 1)
    sh = sh_ref[...].reshape(-1, 1, 1)
    z = y_ref[0, 0].astype(jnp.float32) * sc + sh
    o_ref[0, :, 0] = jnp.maximum(z, 0.0)


def kernel(x, w, b, gamma, beta, alpha):
    N, Cin, D, H, W = x.shape
    Cout = w.shape[0]
    HW = H * W
    K = 27 * Cin

    w_t = jnp.transpose(w, (0, 3, 4, 2, 1)).reshape(Cout, K).astype(jnp.bfloat16)
    b2 = b.reshape(Cout, 1).astype(jnp.float32)

    def x_spec(shift):
        return pl.BlockSpec(
            (1, Cin, 1, H, W),
            lambda n, d, s=shift: (n, 0, jnp.clip(d + s - 1, 0, D - 1), 0, 0))

    y, s_sum, s_sq = pl.pallas_call(
        functools.partial(_conv_kernel, H=H, W=W, D=D),
        out_shape=(
            jax.ShapeDtypeStruct((N, D, Cout, H, W), jnp.bfloat16),
            jax.ShapeDtypeStruct((N, D, Cout, 1), jnp.float32),
            jax.ShapeDtypeStruct((N, D, Cout, 1), jnp.float32),
        ),
        grid_spec=pltpu.PrefetchScalarGridSpec(
            num_scalar_prefetch=0,
            grid=(N, D),
            in_specs=[
                x_spec(0), x_spec(1), x_spec(2),
                pl.BlockSpec((Cout, K), lambda n, d: (0, 0)),
                pl.BlockSpec((Cout, 1), lambda n, d: (0, 0)),
            ],
            out_specs=(
                pl.BlockSpec((1, 1, Cout, H, W), lambda n, d: (n, d, 0, 0, 0)),
                pl.BlockSpec((1, 1, Cout, 1), lambda n, d: (n, d, 0, 0)),
                pl.BlockSpec((1, 1, Cout, 1), lambda n, d: (n, d, 0, 0)),
            ),
        ),
        compiler_params=pltpu.CompilerParams(
            dimension_semantics=("parallel", "parallel"),
            vmem_limit_bytes=_VMEM_LIMIT,
        ),
    )(x, x, x, w_t, b2)

    count = float(N * D * HW)
    mean = jnp.sum(s_sum, axis=(0, 1, 3)) / count
    var = jnp.maximum(jnp.sum(s_sq, axis=(0, 1, 3)) / count - mean * mean, 0.0)
    inv = gamma.astype(jnp.float32) / jnp.sqrt(var + 1e-5)
    scale = inv.reshape(Cout, 1)
    shift = (beta.astype(jnp.float32) - mean * inv).reshape(Cout, 1)

    out = pl.pallas_call(
        _bn_relu_kernel,
        out_shape=jax.ShapeDtypeStruct((N, Cout, D, H, W), jnp.float32),
        grid_spec=pltpu.PrefetchScalarGridSpec(
            num_scalar_prefetch=0,
            grid=(N, D),
            in_specs=[
                pl.BlockSpec((1, 1, Cout, H, W), lambda n, d: (n, d, 0, 0, 0)),
                pl.BlockSpec((Cout, 1), lambda n, d: (0, 0)),
                pl.BlockSpec((Cout, 1), lambda n, d: (0, 0)),
            ],
            out_specs=pl.BlockSpec((1, Cout, 1, H, W),
                                   lambda n, d: (n, 0, d, 0, 0)),
        ),
        compiler_params=pltpu.CompilerParams(
            dimension_semantics=("parallel", "parallel"),
            vmem_limit_bytes=_VMEM_LIMIT,
        ),
    )(y, scale, shift)
    return out


# speedup vs baseline: 4.8292x; 1.4504x over previous
import functools

import jax
import jax.numpy as jnp
from jax.experimental import pallas as pl
from jax.experimental.pallas import tpu as pltpu

_VMEM_LIMIT = 64 * 1024 * 1024


def _shift_lanes(v, k):
    if k > 0:
        return jnp.concatenate([v[:, k:], jnp.zeros((v.shape[0], k), v.dtype)],
                               axis=1)
    if k < 0:
        return jnp.concatenate([jnp.zeros((v.shape[0], -k), v.dtype),
                                v[:, :k]], axis=1)
    return v


def _taps(x2, W, mask_l, mask_r):
    pieces = []
    for kh in range(3):
        for kw in range(3):
            p = _shift_lanes(x2, (kh - 1) * W + (kw - 1))
            if kw == 0:
                p = p * mask_l
            elif kw == 2:
                p = p * mask_r
            pieces.append(p)
    return jnp.concatenate(pieces, axis=0)


def _conv_kernel(xt_ref, xm_ref, xb_ref, w_ref, wt_ref, wb_ref, b_ref,
                 y_ref, s_ref, q_ref, *, H, W, DB, NB):
    dblk = pl.program_id(1)
    Cin = xt_ref.shape[1]
    HW = H * W
    Cout = b_ref.shape[0]

    col = jax.lax.broadcasted_iota(jnp.int32, (1, HW), 1) % W
    mask_l = jnp.where(col == 0, 0.0, 1.0).astype(jnp.bfloat16)
    mask_r = jnp.where(col == W - 1, 0.0, 1.0).astype(jnp.bfloat16)
    top = jnp.where(dblk > 0, 1.0, 0.0)
    bot = jnp.where(dblk < NB - 1, 1.0, 0.0)

    xall = xm_ref[0].reshape(Cin * DB, HW).astype(jnp.bfloat16)
    pt = _taps(xall, W, mask_l, mask_r)
    acc = jnp.dot(w_ref[...], pt,
                  preferred_element_type=jnp.float32)

    et = (xt_ref[0, :, DB - 1, :] * top).astype(jnp.bfloat16)
    eb = (xb_ref[0, :, 0, :] * bot).astype(jnp.bfloat16)
    eacc_t = jnp.dot(wt_ref[...], _taps(et, W, mask_l, mask_r),
                     preferred_element_type=jnp.float32)
    eacc_b = jnp.dot(wb_ref[...], _taps(eb, W, mask_l, mask_r),
                     preferred_element_type=jnp.float32)

    for di in range(DB):
        a = acc[di * Cout:(di + 1) * Cout]
        if di == 0:
            a = a + eacc_t
        if di == DB - 1:
            a = a + eacc_b
        a = a + b_ref[...]
        y_ref[0, :, di] = a.reshape(Cout, H, W).astype(y_ref.dtype)
        s_ref[0, di] = jnp.sum(a, axis=1, keepdims=True)
        q_ref[0, di] = jnp.sum(a * a, axis=1, keepdims=True)


def _bn_relu_kernel(y_ref, s_ref, q_ref, g_ref, be_ref, o_ref, *, count):
    mean = jnp.sum(s_ref[...], axis=0) / count
    var = jnp.maximum(jnp.sum(q_ref[...], axis=0) / count - mean * mean, 0.0)
    inv = g_ref[...] / jnp.sqrt(var + 1e-5)
    shift = be_ref[...] - mean * inv
    sc = inv.reshape(-1, 1, 1, 1)
    sh = shift.reshape(-1, 1, 1, 1)
    z = y_ref[0].astype(jnp.float32) * sc + sh
    o_ref[0] = jnp.maximum(z, 0.0)


def kernel(x, w, b, gamma, beta, alpha):
    N, Cin, D, H, W = x.shape
    Cout = w.shape[0]
    HW = H * W
    DB = 8
    while D % DB != 0:
        DB //= 2
    NB = D // DB
    K = 9 * Cin * DB

    x4 = x.reshape(N, Cin, D, HW)
    wt = jnp.transpose(w, (0, 3, 4, 1, 2)).reshape(Cout, 9, Cin, 3)
    ar = jnp.arange(DB)
    eye = (ar[:, None, None] + jnp.arange(3)[None, None, :] - 1
           == ar[None, :, None]).astype(w.dtype)
    w_all = jnp.einsum('ctik,dek->dctie', wt, eye)
    w_all = w_all.reshape(DB * Cout, K).astype(jnp.bfloat16)
    w_top = wt[:, :, :, 0].reshape(Cout, 9 * Cin).astype(jnp.bfloat16)
    w_bot = wt[:, :, :, 2].reshape(Cout, 9 * Cin).astype(jnp.bfloat16)
    b2 = b.reshape(Cout, 1).astype(jnp.float32)

    def x_spec(shift):
        return pl.BlockSpec(
            (1, Cin, DB, HW),
            lambda n, i, s=shift: (n, 0, jnp.clip(i + s, 0, NB - 1), 0))

    y, s_sum, s_sq = pl.pallas_call(
        functools.partial(_conv_kernel, H=H, W=W, DB=DB, NB=NB),
        out_shape=(
            jax.ShapeDtypeStruct((N, Cout, D, H, W), jnp.bfloat16),
            jax.ShapeDtypeStruct((N, D, Cout, 1), jnp.float32),
            jax.ShapeDtypeStruct((N, D, Cout, 1), jnp.float32),
        ),
        grid_spec=pltpu.PrefetchScalarGridSpec(
            num_scalar_prefetch=0,
            grid=(N, NB),
            in_specs=[
                x_spec(-1), x_spec(0), x_spec(1),
                pl.BlockSpec((DB * Cout, K), lambda n, i: (0, 0)),
                pl.BlockSpec((Cout, 9 * Cin), lambda n, i: (0, 0)),
                pl.BlockSpec((Cout, 9 * Cin), lambda n, i: (0, 0)),
                pl.BlockSpec((Cout, 1), lambda n, i: (0, 0)),
            ],
            out_specs=(
                pl.BlockSpec((1, Cout, DB, H, W), lambda n, i: (n, 0, i, 0, 0)),
                pl.BlockSpec((1, DB, Cout, 1), lambda n, i: (n, i, 0, 0)),
                pl.BlockSpec((1, DB, Cout, 1), lambda n, i: (n, i, 0, 0)),
            ),
        ),
        compiler_params=pltpu.CompilerParams(
            dimension_semantics=("parallel", "parallel"),
            vmem_limit_bytes=_VMEM_LIMIT,
        ),
    )(x4, x4, x4, w_all, w_top, w_bot, b2)

    out = pl.pallas_call(
        functools.partial(_bn_relu_kernel, count=float(N * D * HW)),
        out_shape=jax.ShapeDtypeStruct((N, Cout, D, H, W), jnp.float32),
        grid_spec=pltpu.PrefetchScalarGridSpec(
            num_scalar_prefetch=0,
            grid=(N, NB),
            in_specs=[
                pl.BlockSpec((1, Cout, DB, H, W), lambda n, i: (n, 0, i, 0, 0)),
                pl.BlockSpec((N * D, Cout, 1), lambda n, i: (0, 0, 0)),
                pl.BlockSpec((N * D, Cout, 1), lambda n, i: (0, 0, 0)),
                pl.BlockSpec((Cout, 1), lambda n, i: (0, 0)),
                pl.BlockSpec((Cout, 1), lambda n, i: (0, 0)),
            ],
            out_specs=pl.BlockSpec((1, Cout, DB, H, W),
                                   lambda n, i: (n, 0, i, 0, 0)),
        ),
        compiler_params=pltpu.CompilerParams(
            dimension_semantics=("parallel", "parallel"),
            vmem_limit_bytes=_VMEM_LIMIT,
        ),
    )(y, s_sum.reshape(N * D, Cout, 1), s_sq.reshape(N * D, Cout, 1),
      gamma.reshape(Cout, 1).astype(jnp.float32),
      beta.reshape(Cout, 1).astype(jnp.float32))
    return out
